# Initial kernel scaffold; baseline (speedup 1.0000x reference)
import functools

import jax
import jax.numpy as jnp
from jax import lax
from jax.experimental import pallas as pl
from jax.experimental.pallas import tpu as pltpu
from jax.experimental.pallas import tpu_sc as plsc

N = 10000
E = 320000
D = 128
G = 64
EPS = 1e-5

NC = 2
NS = 16
LANES = 16
CHUNK = 80
EPT = E // (NC * NS)
STEPS = EPT // CHUNK
RPT = N // NS


def _edge_pass_body(with_deg, *refs):
    if with_deg:
        (h_hbm, src_hbm, dst_hbm, agg_out, deg_out,
         idx_src, idx_dst, rows, deg_local, agg_shared, sem) = refs
    else:
        (h_hbm, src_hbm, dst_hbm, agg_out,
         idx_src, idx_dst, rows, agg_shared, sem) = refs

    cid = lax.axis_index("c")
    sid = lax.axis_index("s")

    zeros16 = jnp.zeros((LANES,), jnp.float32)

    def zrow(i, carry):
        rows[i // (D // LANES), pl.ds((i % (D // LANES)) * LANES, LANES)] = zeros16
        return carry
    lax.fori_loop(0, CHUNK * (D // LANES), zrow, 0)

    for j in range(RPT // CHUNK):
        pltpu.sync_copy(rows, agg_shared.at[pl.ds(sid * RPT + j * CHUNK, CHUNK)])
    rem = RPT % CHUNK
    if rem:
        pltpu.sync_copy(rows.at[pl.ds(0, rem)],
                        agg_shared.at[pl.ds(sid * RPT + (RPT // CHUNK) * CHUNK, rem)])

    if with_deg:
        def zdeg(i, carry):
            deg_local[pl.ds(i * LANES, LANES)] = zeros16
            return carry
        lax.fori_loop(0, N // LANES, zdeg, 0)

    plsc.subcore_barrier()

    ones16 = jnp.ones((LANES,), jnp.float32)
    base0 = cid * (NS * EPT) + sid * EPT

    def step(i, carry):
        base = base0 + i * CHUNK
        pltpu.sync_copy(src_hbm.at[pl.ds(base, CHUNK)], idx_src)
        pltpu.sync_copy(dst_hbm.at[pl.ds(base, CHUNK)], idx_dst)
        pltpu.async_copy(h_hbm.at[idx_src], rows, sem).wait()
        pltpu.sync_copy(rows, agg_shared.at[idx_dst], add=True)
        if with_deg:
            for j in range(CHUNK // LANES):
                d = idx_dst[pl.ds(j * LANES, LANES)]
                plsc.addupdate_scatter(deg_local, [d], ones16)
        return carry
    lax.fori_loop(0, STEPS, step, 0)

    plsc.subcore_barrier()

    pltpu.sync_copy(agg_shared.at[pl.ds(sid * RPT, RPT)],
                    agg_out.at[cid, pl.ds(sid * RPT, RPT)])
    if with_deg:
        pltpu.sync_copy(deg_local, deg_out.at[cid * NS + sid])


def _make_edge_pass(with_deg):
    out_type = [jax.ShapeDtypeStruct((NC, N, D), jnp.float32)]
    if with_deg:
        out_type.append(jax.ShapeDtypeStruct((NC * NS, N), jnp.float32))
    scratch = [
        pltpu.VMEM((CHUNK,), jnp.int32),
        pltpu.VMEM((CHUNK,), jnp.int32),
        pltpu.VMEM((CHUNK, D), jnp.float32),
    ]
    if with_deg:
        scratch.append(pltpu.VMEM((N,), jnp.float32))
    scratch += [
        pltpu.VMEM_SHARED((N, D), jnp.float32),
        pltpu.SemaphoreType.DMA,
    ]
    mesh = plsc.VectorSubcoreMesh(core_axis_name="c", subcore_axis_name="s")
    return pl.kernel(
        functools.partial(_edge_pass_body, with_deg),
        out_type=tuple(out_type),
        mesh=mesh,
        scratch_types=tuple(scratch),
    )


_edge_pass_deg = _make_edge_pass(True)
_edge_pass = _make_edge_pass(False)


def _inv_deg_full(deg32):
    deg = jnp.sum(deg32, axis=0, keepdims=True)
    inv = 1.0 / jnp.maximum(deg, 1.0)
    ones_r = jnp.ones((1, D), jnp.float32)
    return lax.dot_general(inv, ones_r, (((0,), (0,)), ((), ())),
                           preferred_element_type=jnp.float32)


def _layer_math(hprev, agg2, deg32, W_l, b_l, W_r, gamma, beta):
    agg = agg2[0] + agg2[1]
    mean = agg * _inv_deg_full(deg32)
    pre = (jnp.dot(mean, W_l, preferred_element_type=jnp.float32)
           + jnp.dot(hprev, W_r, preferred_element_type=jnp.float32)
           + b_l)
    mu = jnp.mean(pre, axis=0, keepdims=True)
    cen = pre - mu
    var = jnp.mean(cen * cen, axis=0, keepdims=True)
    return jnp.maximum(cen * lax.rsqrt(var + EPS) * gamma + beta, 0.0)


def _layer0_body(h_ref, agg_ref, deg_ref, wl_ref, bl_ref, wr_ref, g_ref, be_ref,
                 out_ref):
    out_ref[...] = _layer_math(h_ref[...], agg_ref[...], deg_ref[...],
                               wl_ref[...], bl_ref[...], wr_ref[...],
                               g_ref[...], be_ref[...])


def _layer1_body(h_ref, agg_ref, deg_ref, wl_ref, bl_ref, wr_ref, g_ref, be_ref,
                 batch_ref, out_ref):
    h2 = _layer_math(h_ref[...], agg_ref[...], deg_ref[...],
                     wl_ref[...], bl_ref[...], wr_ref[...],
                     g_ref[...], be_ref[...])
    gids = lax.broadcasted_iota(jnp.int32, (G, N), 0)
    onehot = jnp.where(gids == batch_ref[...], 1.0, 0.0)
    out_ref[...] = lax.dot_general(onehot, h2, (((1,), (0,)), ((), ())),
                                   preferred_element_type=jnp.float32)


_layer0 = pl.pallas_call(
    _layer0_body,
    out_shape=jax.ShapeDtypeStruct((N, D), jnp.float32),
)

_layer1 = pl.pallas_call(
    _layer1_body,
    out_shape=jax.ShapeDtypeStruct((G, D), jnp.float32),
)


def kernel(x, edge_index, batch, W_l0, b_l0, W_r0, gamma0, beta0,
           W_l1, b_l1, W_r1, gamma1, beta1):
    src = edge_index[0]
    dst = edge_index[1]
    batch2d = batch.reshape(1, N)

    agg0, deg32 = _edge_pass_deg(x, src, dst)
    h1 = _layer0(x, agg0, deg32, W_l0, b_l0.reshape(1, D), W_r0,
                 gamma0.reshape(1, D), beta0.reshape(1, D))
    (agg1,) = _edge_pass(h1, src, dst)
    out = _layer1(h1, agg1, deg32, W_l1, b_l1.reshape(1, D), W_r1,
                  gamma1.reshape(1, D), beta1.reshape(1, D), batch2d)
    return out


# speedup vs baseline: 4.9033x; 4.9033x over previous
import jax
import jax.numpy as jnp
from jax import lax
from jax.experimental import pallas as pl
from jax.experimental.pallas import tpu as pltpu
from jax.experimental.pallas import tpu_sc as plsc

N = 10000
E = 320000
D = 128
G = 64
EPS = 1e-5

NC = 2
NS = 16
LANES = 16
CHUNK = 80
EPT = E // (NC * NS)
STEPS = EPT // CHUNK
RPT = 624
TAIL = N - NS * RPT
REM = RPT % CHUNK


def _fill_buf(buf, vec):
    def body(i, carry):
        for c in range(D // LANES):
            buf[i, pl.ds(c * LANES, LANES)] = vec
        return carry
    lax.fori_loop(0, CHUNK, body, 0)


def _zero_shared(sid, buf, shared):
    for j in range(RPT // CHUNK):
        pltpu.sync_copy(buf, shared.at[pl.ds(sid * RPT + j * CHUNK, CHUNK)])
    if REM:
        pltpu.sync_copy(buf.at[pl.ds(0, REM)],
                        shared.at[pl.ds(sid * RPT + (RPT // CHUNK) * CHUNK, REM)])

    @pl.when(sid == NS - 1)
    def _zero_tail():
        pltpu.sync_copy(buf.at[pl.ds(0, TAIL)], shared.at[pl.ds(NS * RPT, TAIL)])


def _export_shared(cid, sid, shared, out_hbm):
    pltpu.sync_copy(shared.at[pl.ds(sid * RPT, RPT)],
                    out_hbm.at[cid, pl.ds(sid * RPT, RPT)])

    @pl.when(sid == NS - 1)
    def _export_tail():
        pltpu.sync_copy(shared.at[pl.ds(NS * RPT, TAIL)],
                        out_hbm.at[cid, pl.ds(NS * RPT, TAIL)])


def _agg_body(h_hbm, src_hbm, dst_hbm, agg_out, idx_src, idx_dst, rows,
              agg_shared, sem):
    cid = lax.axis_index("c")
    sid = lax.axis_index("s")

    _fill_buf(rows, jnp.zeros((LANES,), jnp.float32))
    _zero_shared(sid, rows, agg_shared)
    plsc.subcore_barrier()

    base0 = cid * (NS * EPT) + sid * EPT

    def step(i, carry):
        base = base0 + i * CHUNK
        pltpu.sync_copy(src_hbm.at[pl.ds(base, CHUNK)], idx_src)
        pltpu.sync_copy(dst_hbm.at[pl.ds(base, CHUNK)], idx_dst)
        pltpu.async_copy(h_hbm.at[idx_src], rows, sem).wait()
        pltpu.sync_copy(rows, agg_shared.at[idx_dst], add=True)
        return carry
    lax.fori_loop(0, STEPS, step, 0)

    plsc.subcore_barrier()
    _export_shared(cid, sid, agg_shared, agg_out)


def _deg_body(dst_hbm, deg_out, idx_dst, ones_rows, deg_shared, sem):
    cid = lax.axis_index("c")
    sid = lax.axis_index("s")

    _fill_buf(ones_rows, jnp.zeros((LANES,), jnp.float32))
    _zero_shared(sid, ones_rows, deg_shared)
    _fill_buf(ones_rows, jnp.ones((LANES,), jnp.float32))
    plsc.subcore_barrier()

    base0 = cid * (NS * EPT) + sid * EPT

    def step(i, carry):
        pltpu.sync_copy(dst_hbm.at[pl.ds(base0 + i * CHUNK, CHUNK)], idx_dst)
        pltpu.sync_copy(ones_rows, deg_shared.at[idx_dst], add=True)
        return carry
    lax.fori_loop(0, STEPS, step, 0)

    plsc.subcore_barrier()
    _export_shared(cid, sid, deg_shared, deg_out)


_SC_MESH = plsc.VectorSubcoreMesh(core_axis_name="c", subcore_axis_name="s")

_agg_pass = pl.kernel(
    _agg_body,
    out_type=jax.ShapeDtypeStruct((NC, N, D), jnp.float32),
    mesh=_SC_MESH,
    scratch_types=(
        pltpu.VMEM((CHUNK,), jnp.int32),
        pltpu.VMEM((CHUNK,), jnp.int32),
        pltpu.VMEM((CHUNK, D), jnp.float32),
        pltpu.VMEM_SHARED((N, D), jnp.float32),
        pltpu.SemaphoreType.DMA,
    ),
)

_deg_pass = pl.kernel(
    _deg_body,
    out_type=jax.ShapeDtypeStruct((NC, N, D), jnp.float32),
    mesh=_SC_MESH,
    scratch_types=(
        pltpu.VMEM((CHUNK,), jnp.int32),
        pltpu.VMEM((CHUNK, D), jnp.float32),
        pltpu.VMEM_SHARED((N, D), jnp.float32),
        pltpu.SemaphoreType.DMA,
    ),
)


def _layer_math(hprev, agg2, deg2, W_l, b_l, W_r, gamma, beta):
    agg = agg2[0] + agg2[1]
    deg = deg2[0] + deg2[1]
    mean = agg / jnp.maximum(deg, 1.0)
    pre = (jnp.dot(mean, W_l, preferred_element_type=jnp.float32)
           + jnp.dot(hprev, W_r, preferred_element_type=jnp.float32)
           + b_l)
    mu = jnp.mean(pre, axis=0, keepdims=True)
    cen = pre - mu
    var = jnp.mean(cen * cen, axis=0, keepdims=True)
    return jnp.maximum(cen * lax.rsqrt(var + EPS) * gamma + beta, 0.0)


def _layer0_body(h_ref, agg_ref, deg_ref, wl_ref, bl_ref, wr_ref, g_ref, be_ref,
                 out_ref):
    out_ref[...] = _layer_math(h_ref[...], agg_ref[...], deg_ref[...],
                               wl_ref[...], bl_ref[...], wr_ref[...],
                               g_ref[...], be_ref[...])


def _layer1_body(h_ref, agg_ref, deg_ref, wl_ref, bl_ref, wr_ref, g_ref, be_ref,
                 batch_ref, out_ref):
    h2 = _layer_math(h_ref[...], agg_ref[...], deg_ref[...],
                     wl_ref[...], bl_ref[...], wr_ref[...],
                     g_ref[...], be_ref[...])
    gids = lax.broadcasted_iota(jnp.int32, (G, N), 0)
    onehot = jnp.where(gids == batch_ref[...], 1.0, 0.0)
    out_ref[...] = lax.dot_general(onehot, h2, (((1,), (0,)), ((), ())),
                                   preferred_element_type=jnp.float32)


_layer0 = pl.pallas_call(
    _layer0_body,
    out_shape=jax.ShapeDtypeStruct((N, D), jnp.float32),
)

_layer1 = pl.pallas_call(
    _layer1_body,
    out_shape=jax.ShapeDtypeStruct((G, D), jnp.float32),
)


def kernel(x, edge_index, batch, W_l0, b_l0, W_r0, gamma0, beta0,
           W_l1, b_l1, W_r1, gamma1, beta1):
    src = edge_index[0]
    dst = edge_index[1]
    batch2d = batch.reshape(1, N)

    deg2 = _deg_pass(dst)
    agg0 = _agg_pass(x, src, dst)
    h1 = _layer0(x, agg0, deg2, W_l0, b_l0.reshape(1, D), W_r0,
                 gamma0.reshape(1, D), beta0.reshape(1, D))
    agg1 = _agg_pass(h1, src, dst)
    out = _layer1(h1, agg1, deg2, W_l1, b_l1.reshape(1, D), W_r1,
                  gamma1.reshape(1, D), beta1.reshape(1, D), batch2d)
    return out


# speedup vs baseline: 8.2781x; 1.6883x over previous
import jax
import jax.numpy as jnp
from jax import lax
from jax.experimental import pallas as pl
from jax.experimental.pallas import tpu as pltpu
from jax.experimental.pallas import tpu_sc as plsc

N = 10000
E = 320000
D = 128
G = 64
EPS = 1e-5

NC = 2
NS = 16
LANES = 16
CHUNK = 80
EPT = E // (NC * NS)
STEPS = EPT // CHUNK
RPT = 624
TAIL = N - NS * RPT
REM = RPT % CHUNK


def _fill_buf(buf, vec):
    def body(i, carry):
        for c in range(D // LANES):
            buf[i, pl.ds(c * LANES, LANES)] = vec
        return carry
    lax.fori_loop(0, CHUNK, body, 0)


def _zero_shared(sid, buf, shared):
    for j in range(RPT // CHUNK):
        pltpu.sync_copy(buf, shared.at[pl.ds(sid * RPT + j * CHUNK, CHUNK)])
    if REM:
        pltpu.sync_copy(buf.at[pl.ds(0, REM)],
                        shared.at[pl.ds(sid * RPT + (RPT // CHUNK) * CHUNK, REM)])

    @pl.when(sid == NS - 1)
    def _zero_tail():
        pltpu.sync_copy(buf.at[pl.ds(0, TAIL)], shared.at[pl.ds(NS * RPT, TAIL)])


def _export_shared(cid, sid, shared, out_hbm):
    pltpu.sync_copy(shared.at[pl.ds(sid * RPT, RPT)],
                    out_hbm.at[cid, pl.ds(sid * RPT, RPT)])

    @pl.when(sid == NS - 1)
    def _export_tail():
        pltpu.sync_copy(shared.at[pl.ds(NS * RPT, TAIL)],
                        out_hbm.at[cid, pl.ds(NS * RPT, TAIL)])


NBUF = 4
QUADS = STEPS // NBUF
REM_STEPS = STEPS % NBUF


def _agg_body(h_hbm, src_hbm, dst_hbm, agg_out,
              s0b, s1b, s2b, s3b, d0, d1, d2, d3, r0, r1, r2, r3,
              agg_shared, ss0, ss1, ss2, ss3, si0, si1, si2, si3,
              sg0, sg1, sg2, sg3):
    cid = lax.axis_index("c")
    sid = lax.axis_index("s")
    base0 = cid * (NS * EPT) + sid * EPT

    _fill_buf(r0, jnp.zeros((LANES,), jnp.float32))
    _zero_shared(sid, r0, agg_shared)
    plsc.subcore_barrier()

    sbufs = (s0b, s1b, s2b, s3b)
    dbufs = (d0, d1, d2, d3)
    rbufs = (r0, r1, r2, r3)
    ssems = (ss0, ss1, ss2, ss3)
    isems = (si0, si1, si2, si3)
    gsems = (sg0, sg1, sg2, sg3)

    def idx_start(off, k):
        ds_ = pltpu.async_copy(src_hbm.at[pl.ds(base0 + off, CHUNK)],
                               sbufs[k], ssems[k])
        dd_ = pltpu.async_copy(dst_hbm.at[pl.ds(base0 + off, CHUNK)],
                               dbufs[k], isems[k])
        return ds_, dd_

    def quad(q, carry):
        qb = q * (NBUF * CHUNK)
        idescs = [idx_start(qb + k * CHUNK, k) for k in range(NBUF)]
        gdescs = []
        for k in range(NBUF):
            idescs[k][0].wait()
            gdescs.append(pltpu.async_copy(h_hbm.at[sbufs[k]], rbufs[k],
                                           gsems[k]))
        for k in range(NBUF):
            gdescs[k].wait()
            idescs[k][1].wait()
            pltpu.sync_copy(rbufs[k], agg_shared.at[dbufs[k]], add=True)
        return carry
    lax.fori_loop(0, QUADS, quad, 0)

    for k in range(REM_STEPS):
        off = (QUADS * NBUF + k) * CHUNK
        ds_, dd_ = idx_start(off, k)
        ds_.wait()
        pltpu.async_copy(h_hbm.at[sbufs[k]], rbufs[k], gsems[k]).wait()
        dd_.wait()
        pltpu.sync_copy(rbufs[k], agg_shared.at[dbufs[k]], add=True)

    plsc.subcore_barrier()
    _export_shared(cid, sid, agg_shared, agg_out)


def _deg_body(dst_hbm, deg_out, d0, d1, ones_rows, deg_shared, si0, si1):
    cid = lax.axis_index("c")
    sid = lax.axis_index("s")
    base0 = cid * (NS * EPT) + sid * EPT

    _fill_buf(ones_rows, jnp.zeros((LANES,), jnp.float32))
    _zero_shared(sid, ones_rows, deg_shared)
    _fill_buf(ones_rows, jnp.ones((LANES,), jnp.float32))
    plsc.subcore_barrier()

    dbufs = (d0, d1)
    isems = (si0, si1)

    def duo(q, carry):
        qb = q * (2 * CHUNK)
        descs = [pltpu.async_copy(
            dst_hbm.at[pl.ds(base0 + qb + k * CHUNK, CHUNK)],
            dbufs[k], isems[k]) for k in range(2)]
        for k in range(2):
            descs[k].wait()
            pltpu.sync_copy(ones_rows, deg_shared.at[dbufs[k]], add=True)
        return carry
    lax.fori_loop(0, STEPS // 2, duo, 0)

    for k in range(STEPS % 2):
        off = ((STEPS // 2) * 2 + k) * CHUNK
        pltpu.async_copy(dst_hbm.at[pl.ds(base0 + off, CHUNK)],
                         d0, si0).wait()
        pltpu.sync_copy(ones_rows, deg_shared.at[d0], add=True)

    plsc.subcore_barrier()
    _export_shared(cid, sid, deg_shared, deg_out)


_SC_MESH = plsc.VectorSubcoreMesh(core_axis_name="c", subcore_axis_name="s")

_agg_pass = pl.kernel(
    _agg_body,
    out_type=jax.ShapeDtypeStruct((NC, N, D), jnp.float32),
    mesh=_SC_MESH,
    scratch_types=(
        pltpu.VMEM((CHUNK,), jnp.int32),
        pltpu.VMEM((CHUNK,), jnp.int32),
        pltpu.VMEM((CHUNK,), jnp.int32),
        pltpu.VMEM((CHUNK,), jnp.int32),
        pltpu.VMEM((CHUNK,), jnp.int32),
        pltpu.VMEM((CHUNK,), jnp.int32),
        pltpu.VMEM((CHUNK,), jnp.int32),
        pltpu.VMEM((CHUNK,), jnp.int32),
        pltpu.VMEM((CHUNK, D), jnp.float32),
        pltpu.VMEM((CHUNK, D), jnp.float32),
        pltpu.VMEM((CHUNK, D), jnp.float32),
        pltpu.VMEM((CHUNK, D), jnp.float32),
        pltpu.VMEM_SHARED((N, D), jnp.float32),
        pltpu.SemaphoreType.DMA,
        pltpu.SemaphoreType.DMA,
        pltpu.SemaphoreType.DMA,
        pltpu.SemaphoreType.DMA,
        pltpu.SemaphoreType.DMA,
        pltpu.SemaphoreType.DMA,
        pltpu.SemaphoreType.DMA,
        pltpu.SemaphoreType.DMA,
        pltpu.SemaphoreType.DMA,
        pltpu.SemaphoreType.DMA,
        pltpu.SemaphoreType.DMA,
        pltpu.SemaphoreType.DMA,
    ),
)

_deg_pass = pl.kernel(
    _deg_body,
    out_type=jax.ShapeDtypeStruct((NC, N, D), jnp.float32),
    mesh=_SC_MESH,
    scratch_types=(
        pltpu.VMEM((CHUNK,), jnp.int32),
        pltpu.VMEM((CHUNK,), jnp.int32),
        pltpu.VMEM((CHUNK, D), jnp.float32),
        pltpu.VMEM_SHARED((N, D), jnp.float32),
        pltpu.SemaphoreType.DMA,
        pltpu.SemaphoreType.DMA,
    ),
)


def _layer_math(hprev, agg2, deg2, W_l, b_l, W_r, gamma, beta):
    agg = agg2[0] + agg2[1]
    deg = deg2[0] + deg2[1]
    mean = agg / jnp.maximum(deg, 1.0)
    pre = (jnp.dot(mean, W_l, preferred_element_type=jnp.float32)
           + jnp.dot(hprev, W_r, preferred_element_type=jnp.float32)
           + b_l)
    mu = jnp.mean(pre, axis=0, keepdims=True)
    cen = pre - mu
    var = jnp.mean(cen * cen, axis=0, keepdims=True)
    return jnp.maximum(cen * lax.rsqrt(var + EPS) * gamma + beta, 0.0)


def _layer0_body(h_ref, agg_ref, deg_ref, wl_ref, bl_ref, wr_ref, g_ref, be_ref,
                 out_ref):
    out_ref[...] = _layer_math(h_ref[...], agg_ref[...], deg_ref[...],
                               wl_ref[...], bl_ref[...], wr_ref[...],
                               g_ref[...], be_ref[...])


def _layer1_body(h_ref, agg_ref, deg_ref, wl_ref, bl_ref, wr_ref, g_ref, be_ref,
                 batch_ref, out_ref):
    h2 = _layer_math(h_ref[...], agg_ref[...], deg_ref[...],
                     wl_ref[...], bl_ref[...], wr_ref[...],
                     g_ref[...], be_ref[...])
    gids = lax.broadcasted_iota(jnp.int32, (G, N), 0)
    onehot = jnp.where(gids == batch_ref[...], 1.0, 0.0)
    out_ref[...] = lax.dot_general(onehot, h2, (((1,), (0,)), ((), ())),
                                   preferred_element_type=jnp.float32)


_layer0 = pl.pallas_call(
    _layer0_body,
    out_shape=jax.ShapeDtypeStruct((N, D), jnp.float32),
)

_layer1 = pl.pallas_call(
    _layer1_body,
    out_shape=jax.ShapeDtypeStruct((G, D), jnp.float32),
)


def kernel(x, edge_index, batch, W_l0, b_l0, W_r0, gamma0, beta0,
           W_l1, b_l1, W_r1, gamma1, beta1):
    src = edge_index[0]
    dst = edge_index[1]
    batch2d = batch.reshape(1, N)

    deg2 = _deg_pass(dst)
    agg0 = _agg_pass(x, src, dst)
    h1 = _layer0(x, agg0, deg2, W_l0, b_l0.reshape(1, D), W_r0,
                 gamma0.reshape(1, D), beta0.reshape(1, D))
    agg1 = _agg_pass(h1, src, dst)
    out = _layer1(h1, agg1, deg2, W_l1, b_l1.reshape(1, D), W_r1,
                  gamma1.reshape(1, D), beta1.reshape(1, D), batch2d)
    return out


# speedup vs baseline: 9.5384x; 1.1523x over previous
import jax
import jax.numpy as jnp
from jax import lax
from jax.experimental import pallas as pl
from jax.experimental.pallas import tpu as pltpu
from jax.experimental.pallas import tpu_sc as plsc

N = 10000
E = 320000
D = 128
G = 64
EPS = 1e-5

NC = 2
NS = 16
LANES = 16
CHUNK = 80
EPT = E // (NC * NS)
STEPS = EPT // CHUNK
RPT = 624
TAIL = N - NS * RPT
REM = RPT % CHUNK


def _fill_buf(buf, vec):
    def body(i, carry):
        for c in range(D // LANES):
            buf[i, pl.ds(c * LANES, LANES)] = vec
        return carry
    lax.fori_loop(0, CHUNK, body, 0)


def _zero_shared(sid, buf, shared):
    for j in range(RPT // CHUNK):
        pltpu.sync_copy(buf, shared.at[pl.ds(sid * RPT + j * CHUNK, CHUNK)])
    if REM:
        pltpu.sync_copy(buf.at[pl.ds(0, REM)],
                        shared.at[pl.ds(sid * RPT + (RPT // CHUNK) * CHUNK, REM)])

    @pl.when(sid == NS - 1)
    def _zero_tail():
        pltpu.sync_copy(buf.at[pl.ds(0, TAIL)], shared.at[pl.ds(NS * RPT, TAIL)])


def _export_shared(cid, sid, shared, out_hbm):
    pltpu.sync_copy(shared.at[pl.ds(sid * RPT, RPT)],
                    out_hbm.at[cid, pl.ds(sid * RPT, RPT)])

    @pl.when(sid == NS - 1)
    def _export_tail():
        pltpu.sync_copy(shared.at[pl.ds(NS * RPT, TAIL)],
                        out_hbm.at[cid, pl.ds(NS * RPT, TAIL)])


NBUF = 4
QUADS = STEPS // NBUF
REM_STEPS = STEPS % NBUF


def _agg_body(h_hbm, src_hbm, dst_hbm, agg_out, *sc):
    (s0b, s1b, s2b, s3b, d0, d1, d2, d3, r0, r1, r2, r3, agg_shared,
     ss0, ss1, ss2, ss3, si0, si1, si2, si3, sg0, sg1, sg2, sg3) = sc
    sbufs = (s0b, s1b, s2b, s3b)
    dbufs = (d0, d1, d2, d3)
    rbufs = (r0, r1, r2, r3)
    ssems = (ss0, ss1, ss2, ss3)
    isems = (si0, si1, si2, si3)
    gsems = (sg0, sg1, sg2, sg3)
    cid = lax.axis_index("c")
    sid = lax.axis_index("s")
    base0 = cid * (NS * EPT) + sid * EPT

    _fill_buf(r0, jnp.zeros((LANES,), jnp.float32))
    _zero_shared(sid, r0, agg_shared)
    plsc.subcore_barrier()

    def issue_idx(c, k):
        pltpu.async_copy(src_hbm.at[pl.ds(base0 + c * CHUNK, CHUNK)],
                         sbufs[k], ssems[k])
        pltpu.async_copy(dst_hbm.at[pl.ds(base0 + c * CHUNK, CHUNK)],
                         dbufs[k], isems[k])

    def wait_idx_src(k):
        pltpu.make_async_copy(src_hbm.at[pl.ds(0, CHUNK)],
                              sbufs[k], ssems[k]).wait()

    def wait_idx_dst(k):
        pltpu.make_async_copy(dst_hbm.at[pl.ds(0, CHUNK)],
                              dbufs[k], isems[k]).wait()

    def issue_gather(k):
        pltpu.async_copy(h_hbm.at[sbufs[k]], rbufs[k], gsems[k])

    def wait_gather(k):
        pltpu.make_async_copy(h_hbm.at[sbufs[k]], rbufs[k], gsems[k]).wait()

    def scatter(k):
        pltpu.sync_copy(rbufs[k], agg_shared.at[dbufs[k]], add=True)

    for k in range(NBUF):
        issue_idx(k, k)
    for k in range(NBUF):
        wait_idx_src(k)
        issue_gather(k)

    def round_(j, carry):
        c0 = j * NBUF
        for k in range(NBUF):
            wait_gather(k)
            wait_idx_dst(k)
            scatter(k)
            issue_idx(c0 + NBUF + k, k)
        for k in range(NBUF):
            wait_idx_src(k)
            issue_gather(k)
        return carry
    lax.fori_loop(0, QUADS - 1, round_, 0)

    for k in range(NBUF):
        wait_gather(k)
        wait_idx_dst(k)
        scatter(k)
        if k == 0 and REM_STEPS:
            issue_idx(STEPS - 1, 0)
    if REM_STEPS:
        wait_idx_src(0)
        issue_gather(0)
        wait_gather(0)
        wait_idx_dst(0)
        scatter(0)

    plsc.subcore_barrier()
    _export_shared(cid, sid, agg_shared, agg_out)


def _deg_body(dst_hbm, deg_out, *sc):
    (d0, d1, d2, d3, ones_rows, deg_shared, si0, si1, si2, si3) = sc
    dbufs = (d0, d1, d2, d3)
    isems = (si0, si1, si2, si3)
    cid = lax.axis_index("c")
    sid = lax.axis_index("s")
    base0 = cid * (NS * EPT) + sid * EPT

    _fill_buf(ones_rows, jnp.zeros((LANES,), jnp.float32))
    _zero_shared(sid, ones_rows, deg_shared)
    _fill_buf(ones_rows, jnp.ones((LANES,), jnp.float32))
    plsc.subcore_barrier()

    def issue_idx(c, k):
        pltpu.async_copy(dst_hbm.at[pl.ds(base0 + c * CHUNK, CHUNK)],
                         dbufs[k], isems[k])

    def wait_idx(k):
        pltpu.make_async_copy(dst_hbm.at[pl.ds(0, CHUNK)],
                              dbufs[k], isems[k]).wait()

    def scatter(k):
        pltpu.sync_copy(ones_rows, deg_shared.at[dbufs[k]], add=True)

    for k in range(NBUF):
        issue_idx(k, k)

    def round_(j, carry):
        c0 = j * NBUF
        for k in range(NBUF):
            wait_idx(k)
            scatter(k)
            issue_idx(c0 + NBUF + k, k)
        return carry
    lax.fori_loop(0, QUADS - 1, round_, 0)

    for k in range(NBUF):
        wait_idx(k)
        scatter(k)
        if k == 0 and REM_STEPS:
            issue_idx(STEPS - 1, 0)
    if REM_STEPS:
        wait_idx(0)
        scatter(0)

    plsc.subcore_barrier()
    _export_shared(cid, sid, deg_shared, deg_out)


_SC_MESH = plsc.VectorSubcoreMesh(core_axis_name="c", subcore_axis_name="s")

_agg_pass = pl.kernel(
    _agg_body,
    out_type=jax.ShapeDtypeStruct((NC, N, D), jnp.float32),
    mesh=_SC_MESH,
    scratch_types=(
        pltpu.VMEM((CHUNK,), jnp.int32),
        pltpu.VMEM((CHUNK,), jnp.int32),
        pltpu.VMEM((CHUNK,), jnp.int32),
        pltpu.VMEM((CHUNK,), jnp.int32),
        pltpu.VMEM((CHUNK,), jnp.int32),
        pltpu.VMEM((CHUNK,), jnp.int32),
        pltpu.VMEM((CHUNK,), jnp.int32),
        pltpu.VMEM((CHUNK,), jnp.int32),
        pltpu.VMEM((CHUNK, D), jnp.float32),
        pltpu.VMEM((CHUNK, D), jnp.float32),
        pltpu.VMEM((CHUNK, D), jnp.float32),
        pltpu.VMEM((CHUNK, D), jnp.float32),
        pltpu.VMEM_SHARED((N, D), jnp.float32),
        pltpu.SemaphoreType.DMA,
        pltpu.SemaphoreType.DMA,
        pltpu.SemaphoreType.DMA,
        pltpu.SemaphoreType.DMA,
        pltpu.SemaphoreType.DMA,
        pltpu.SemaphoreType.DMA,
        pltpu.SemaphoreType.DMA,
        pltpu.SemaphoreType.DMA,
        pltpu.SemaphoreType.DMA,
        pltpu.SemaphoreType.DMA,
        pltpu.SemaphoreType.DMA,
        pltpu.SemaphoreType.DMA,
    ),
)

_deg_pass = pl.kernel(
    _deg_body,
    out_type=jax.ShapeDtypeStruct((NC, N, D), jnp.float32),
    mesh=_SC_MESH,
    scratch_types=(
        pltpu.VMEM((CHUNK,), jnp.int32),
        pltpu.VMEM((CHUNK,), jnp.int32),
        pltpu.VMEM((CHUNK,), jnp.int32),
        pltpu.VMEM((CHUNK,), jnp.int32),
        pltpu.VMEM((CHUNK, D), jnp.float32),
        pltpu.VMEM_SHARED((N, D), jnp.float32),
        pltpu.SemaphoreType.DMA,
        pltpu.SemaphoreType.DMA,
        pltpu.SemaphoreType.DMA,
        pltpu.SemaphoreType.DMA,
    ),
)


def _layer_math(hprev, agg2, deg2, W_l, b_l, W_r, gamma, beta):
    agg = agg2[0] + agg2[1]
    deg = deg2[0] + deg2[1]
    mean = agg / jnp.maximum(deg, 1.0)
    pre = (jnp.dot(mean, W_l, preferred_element_type=jnp.float32)
           + jnp.dot(hprev, W_r, preferred_element_type=jnp.float32)
           + b_l)
    mu = jnp.mean(pre, axis=0, keepdims=True)
    cen = pre - mu
    var = jnp.mean(cen * cen, axis=0, keepdims=True)
    return jnp.maximum(cen * lax.rsqrt(var + EPS) * gamma + beta, 0.0)


def _layer0_body(h_ref, agg_ref, deg_ref, wl_ref, bl_ref, wr_ref, g_ref, be_ref,
                 out_ref):
    out_ref[...] = _layer_math(h_ref[...], agg_ref[...], deg_ref[...],
                               wl_ref[...], bl_ref[...], wr_ref[...],
                               g_ref[...], be_ref[...])


def _layer1_body(h_ref, agg_ref, deg_ref, wl_ref, bl_ref, wr_ref, g_ref, be_ref,
                 batch_ref, out_ref):
    h2 = _layer_math(h_ref[...], agg_ref[...], deg_ref[...],
                     wl_ref[...], bl_ref[...], wr_ref[...],
                     g_ref[...], be_ref[...])
    gids = lax.broadcasted_iota(jnp.int32, (G, N), 0)
    onehot = jnp.where(gids == batch_ref[...], 1.0, 0.0)
    out_ref[...] = lax.dot_general(onehot, h2, (((1,), (0,)), ((), ())),
                                   preferred_element_type=jnp.float32)


_layer0 = pl.pallas_call(
    _layer0_body,
    out_shape=jax.ShapeDtypeStruct((N, D), jnp.float32),
)

_layer1 = pl.pallas_call(
    _layer1_body,
    out_shape=jax.ShapeDtypeStruct((G, D), jnp.float32),
)


def kernel(x, edge_index, batch, W_l0, b_l0, W_r0, gamma0, beta0,
           W_l1, b_l1, W_r1, gamma1, beta1):
    src = edge_index[0]
    dst = edge_index[1]
    batch2d = batch.reshape(1, N)

    deg2 = _deg_pass(dst)
    agg0 = _agg_pass(x, src, dst)
    h1 = _layer0(x, agg0, deg2, W_l0, b_l0.reshape(1, D), W_r0,
                 gamma0.reshape(1, D), beta0.reshape(1, D))
    agg1 = _agg_pass(h1, src, dst)
    out = _layer1(h1, agg1, deg2, W_l1, b_l1.reshape(1, D), W_r1,
                  gamma1.reshape(1, D), beta1.reshape(1, D), batch2d)
    return out


# speedup vs baseline: 12.0436x; 1.2626x over previous
import jax
import jax.numpy as jnp
from jax import lax
from jax.experimental import pallas as pl
from jax.experimental.pallas import tpu as pltpu
from jax.experimental.pallas import tpu_sc as plsc

N = 10000
E = 320000
D = 128
G = 64
EPS = 1e-5

NC = 2
NS = 16
LANES = 16
CHUNK = 80
EPT = E // (NC * NS)
STEPS = EPT // CHUNK
RPT = 624
TAIL = N - NS * RPT
REM = RPT % CHUNK


def _fill_buf(buf, vec):
    def body(i, carry):
        for c in range(D // LANES):
            buf[i, pl.ds(c * LANES, LANES)] = vec
        return carry
    lax.fori_loop(0, CHUNK, body, 0)


def _zero_shared(sid, buf, shared):
    for j in range(RPT // CHUNK):
        pltpu.sync_copy(buf, shared.at[pl.ds(sid * RPT + j * CHUNK, CHUNK)])
    if REM:
        pltpu.sync_copy(buf.at[pl.ds(0, REM)],
                        shared.at[pl.ds(sid * RPT + (RPT // CHUNK) * CHUNK, REM)])

    @pl.when(sid == NS - 1)
    def _zero_tail():
        pltpu.sync_copy(buf.at[pl.ds(0, TAIL)], shared.at[pl.ds(NS * RPT, TAIL)])


def _export_shared(cid, sid, shared, out_hbm):
    pltpu.sync_copy(shared.at[pl.ds(sid * RPT, RPT)],
                    out_hbm.at[cid, pl.ds(sid * RPT, RPT)])

    @pl.when(sid == NS - 1)
    def _export_tail():
        pltpu.sync_copy(shared.at[pl.ds(NS * RPT, TAIL)],
                        out_hbm.at[cid, pl.ds(NS * RPT, TAIL)])


NBUF = 4
QUADS = STEPS // NBUF
REM_STEPS = STEPS % NBUF


def _agg_phase(h_hbm, src_hbm, dst_hbm, agg_out, cid, sid,
               sbufs, dbufs, rbufs, ssems, isems, gsems, agg_shared):
    base0 = cid * (NS * EPT) + sid * EPT

    _fill_buf(rbufs[0], jnp.zeros((LANES,), jnp.float32))
    _zero_shared(sid, rbufs[0], agg_shared)
    plsc.subcore_barrier()

    def issue_idx(c, k):
        pltpu.async_copy(src_hbm.at[pl.ds(base0 + c * CHUNK, CHUNK)],
                         sbufs[k], ssems[k])
        pltpu.async_copy(dst_hbm.at[pl.ds(base0 + c * CHUNK, CHUNK)],
                         dbufs[k], isems[k])

    def wait_idx_src(k):
        pltpu.make_async_copy(src_hbm.at[pl.ds(0, CHUNK)],
                              sbufs[k], ssems[k]).wait()

    def wait_idx_dst(k):
        pltpu.make_async_copy(dst_hbm.at[pl.ds(0, CHUNK)],
                              dbufs[k], isems[k]).wait()

    def issue_gather(k):
        pltpu.async_copy(h_hbm.at[sbufs[k]], rbufs[k], gsems[k])

    def wait_gather(k):
        pltpu.make_async_copy(h_hbm.at[sbufs[k]], rbufs[k], gsems[k]).wait()

    def scatter(k):
        pltpu.sync_copy(rbufs[k], agg_shared.at[dbufs[k]], add=True)

    for k in range(NBUF):
        issue_idx(k, k)
    for k in range(NBUF):
        wait_idx_src(k)
        issue_gather(k)

    def round_(j, carry):
        c0 = j * NBUF
        for k in range(NBUF):
            wait_gather(k)
            wait_idx_dst(k)
            scatter(k)
            issue_idx(c0 + NBUF + k, k)
            if k > 0:
                wait_idx_src(k - 1)
                issue_gather(k - 1)
        wait_idx_src(NBUF - 1)
        issue_gather(NBUF - 1)
        return carry
    lax.fori_loop(0, QUADS - 1, round_, 0)

    for k in range(NBUF):
        wait_gather(k)
        wait_idx_dst(k)
        scatter(k)
        if k == 0 and REM_STEPS:
            issue_idx(STEPS - 1, 0)
    if REM_STEPS:
        wait_idx_src(0)
        issue_gather(0)
        wait_gather(0)
        wait_idx_dst(0)
        scatter(0)

    plsc.subcore_barrier()
    _export_shared(cid, sid, agg_shared, agg_out)


def _deg_phase(dst_hbm, deg_out, cid, sid, dbufs, isems, ones_buf,
               deg_shared):
    base0 = cid * (NS * EPT) + sid * EPT

    def issue_idx(c, k):
        pltpu.async_copy(dst_hbm.at[pl.ds(base0 + c * CHUNK, CHUNK)],
                         dbufs[k], isems[k])

    def wait_idx(k):
        pltpu.make_async_copy(dst_hbm.at[pl.ds(0, CHUNK)],
                              dbufs[k], isems[k]).wait()

    def scatter(k):
        pltpu.sync_copy(ones_buf, deg_shared.at[dbufs[k]], add=True)

    for k in range(NBUF):
        issue_idx(k, k)

    def round_(j, carry):
        c0 = j * NBUF
        for k in range(NBUF):
            wait_idx(k)
            scatter(k)
            issue_idx(c0 + NBUF + k, k)
        return carry
    lax.fori_loop(0, QUADS - 1, round_, 0)

    for k in range(NBUF):
        wait_idx(k)
        scatter(k)
        if k == 0 and REM_STEPS:
            issue_idx(STEPS - 1, 0)
    if REM_STEPS:
        wait_idx(0)
        scatter(0)

    plsc.subcore_barrier()
    _export_shared(cid, sid, deg_shared, deg_out)


def _unpack_sc(sc):
    sbufs = sc[0:4]
    dbufs = sc[4:8]
    rbufs = sc[8:12]
    shared = sc[12]
    ssems = sc[13:17]
    isems = sc[17:21]
    gsems = sc[21:25]
    return sbufs, dbufs, rbufs, shared, ssems, isems, gsems


def _agg_body(h_hbm, src_hbm, dst_hbm, agg_out, *sc):
    sbufs, dbufs, rbufs, shared, ssems, isems, gsems = _unpack_sc(sc)
    cid = lax.axis_index("c")
    sid = lax.axis_index("s")
    _agg_phase(h_hbm, src_hbm, dst_hbm, agg_out, cid, sid,
               sbufs, dbufs, rbufs, ssems, isems, gsems, shared)


def _agg_deg_body(h_hbm, src_hbm, dst_hbm, agg_out, deg_out, *sc):
    sbufs, dbufs, rbufs, shared, ssems, isems, gsems = _unpack_sc(sc)
    cid = lax.axis_index("c")
    sid = lax.axis_index("s")
    _agg_phase(h_hbm, src_hbm, dst_hbm, agg_out, cid, sid,
               sbufs, dbufs, rbufs, ssems, isems, gsems, shared)
    plsc.subcore_barrier()
    _fill_buf(rbufs[0], jnp.zeros((LANES,), jnp.float32))
    _zero_shared(sid, rbufs[0], shared)
    _fill_buf(rbufs[0], jnp.ones((LANES,), jnp.float32))
    plsc.subcore_barrier()
    _deg_phase(dst_hbm, deg_out, cid, sid, dbufs, isems, rbufs[0], shared)


_SC_MESH = plsc.VectorSubcoreMesh(core_axis_name="c", subcore_axis_name="s")

_SC_SCRATCH = (
    pltpu.VMEM((CHUNK,), jnp.int32),
    pltpu.VMEM((CHUNK,), jnp.int32),
    pltpu.VMEM((CHUNK,), jnp.int32),
    pltpu.VMEM((CHUNK,), jnp.int32),
    pltpu.VMEM((CHUNK,), jnp.int32),
    pltpu.VMEM((CHUNK,), jnp.int32),
    pltpu.VMEM((CHUNK,), jnp.int32),
    pltpu.VMEM((CHUNK,), jnp.int32),
    pltpu.VMEM((CHUNK, D), jnp.float32),
    pltpu.VMEM((CHUNK, D), jnp.float32),
    pltpu.VMEM((CHUNK, D), jnp.float32),
    pltpu.VMEM((CHUNK, D), jnp.float32),
    pltpu.VMEM_SHARED((N, D), jnp.float32),
) + (pltpu.SemaphoreType.DMA,) * 12

_agg_pass = pl.kernel(
    _agg_body,
    out_type=jax.ShapeDtypeStruct((NC, N, D), jnp.float32),
    mesh=_SC_MESH,
    scratch_types=_SC_SCRATCH,
)

_agg_deg_pass = pl.kernel(
    _agg_deg_body,
    out_type=(jax.ShapeDtypeStruct((NC, N, D), jnp.float32),
              jax.ShapeDtypeStruct((NC, N, D), jnp.float32)),
    mesh=_SC_MESH,
    scratch_types=_SC_SCRATCH,
)


def _layer_math(hprev, agg2, deg2, W_l, b_l, W_r, gamma, beta):
    agg = agg2[0] + agg2[1]
    deg = deg2[0] + deg2[1]
    mean = agg / jnp.maximum(deg, 1.0)
    pre = (jnp.dot(mean, W_l, preferred_element_type=jnp.float32)
           + jnp.dot(hprev, W_r, preferred_element_type=jnp.float32)
           + b_l)
    mu = jnp.mean(pre, axis=0, keepdims=True)
    cen = pre - mu
    var = jnp.mean(cen * cen, axis=0, keepdims=True)
    return jnp.maximum(cen * lax.rsqrt(var + EPS) * gamma + beta, 0.0)


def _layer0_body(h_ref, agg_ref, deg_ref, wl_ref, bl_ref, wr_ref, g_ref, be_ref,
                 out_ref):
    out_ref[...] = _layer_math(h_ref[...], agg_ref[...], deg_ref[...],
                               wl_ref[...], bl_ref[...], wr_ref[...],
                               g_ref[...], be_ref[...])


def _layer1_body(h_ref, agg_ref, deg_ref, wl_ref, bl_ref, wr_ref, g_ref, be_ref,
                 batch_ref, out_ref):
    h2 = _layer_math(h_ref[...], agg_ref[...], deg_ref[...],
                     wl_ref[...], bl_ref[...], wr_ref[...],
                     g_ref[...], be_ref[...])
    gids = lax.broadcasted_iota(jnp.int32, (G, N), 0)
    onehot = jnp.where(gids == batch_ref[...], 1.0, 0.0)
    out_ref[...] = lax.dot_general(onehot, h2, (((1,), (0,)), ((), ())),
                                   preferred_element_type=jnp.float32)


_layer0 = pl.pallas_call(
    _layer0_body,
    out_shape=jax.ShapeDtypeStruct((N, D), jnp.float32),
)

_layer1 = pl.pallas_call(
    _layer1_body,
    out_shape=jax.ShapeDtypeStruct((G, D), jnp.float32),
)


def kernel(x, edge_index, batch, W_l0, b_l0, W_r0, gamma0, beta0,
           W_l1, b_l1, W_r1, gamma1, beta1):
    src = edge_index[0]
    dst = edge_index[1]
    batch2d = batch.reshape(1, N)

    agg0, deg2 = _agg_deg_pass(x, src, dst)
    h1 = _layer0(x, agg0, deg2, W_l0, b_l0.reshape(1, D), W_r0,
                 gamma0.reshape(1, D), beta0.reshape(1, D))
    agg1 = _agg_pass(h1, src, dst)
    out = _layer1(h1, agg1, deg2, W_l1, b_l1.reshape(1, D), W_r1,
                  gamma1.reshape(1, D), beta1.reshape(1, D), batch2d)
    return out


# speedup vs baseline: 12.5845x; 1.0449x over previous
import jax
import jax.numpy as jnp
from jax import lax
from jax.experimental import pallas as pl
from jax.experimental.pallas import tpu as pltpu
from jax.experimental.pallas import tpu_sc as plsc

N = 10000
E = 320000
D = 128
G = 64
EPS = 1e-5

NC = 2
NS = 16
LANES = 16
CHUNK = 80
EPT = E // (NC * NS)
STEPS = EPT // CHUNK
RPT = 624
TAIL = N - NS * RPT
REM = RPT % CHUNK


def _fill_buf(buf, vec):
    def body(i, carry):
        for c in range(D // LANES):
            buf[i, pl.ds(c * LANES, LANES)] = vec
        return carry
    lax.fori_loop(0, CHUNK, body, 0)


def _zero_shared(sid, buf, shared):
    for j in range(RPT // CHUNK):
        pltpu.sync_copy(buf, shared.at[pl.ds(sid * RPT + j * CHUNK, CHUNK)])
    if REM:
        pltpu.sync_copy(buf.at[pl.ds(0, REM)],
                        shared.at[pl.ds(sid * RPT + (RPT // CHUNK) * CHUNK, REM)])

    @pl.when(sid == NS - 1)
    def _zero_tail():
        pltpu.sync_copy(buf.at[pl.ds(0, TAIL)], shared.at[pl.ds(NS * RPT, TAIL)])


def _export_shared(cid, sid, shared, out_hbm):
    pltpu.sync_copy(shared.at[pl.ds(sid * RPT, RPT)],
                    out_hbm.at[cid, pl.ds(sid * RPT, RPT)])

    @pl.when(sid == NS - 1)
    def _export_tail():
        pltpu.sync_copy(shared.at[pl.ds(NS * RPT, TAIL)],
                        out_hbm.at[cid, pl.ds(NS * RPT, TAIL)])


NBUF = 4
QUADS = STEPS // NBUF
REM_STEPS = STEPS % NBUF


def _agg_phase(h_hbm, src_hbm, dst_hbm, agg_out, cid, sid,
               sbufsA, sbufsB, dbufsA, dbufsB, rbufs,
               ssemsA, ssemsB, isemsA, isemsB, gsems, agg_shared):
    base0 = cid * (NS * EPT) + sid * EPT

    _fill_buf(rbufs[0], jnp.zeros((LANES,), jnp.float32))
    _zero_shared(sid, rbufs[0], agg_shared)
    plsc.subcore_barrier()

    def issue_idx(c, sb, db, ss, si, k):
        pltpu.async_copy(src_hbm.at[pl.ds(base0 + c * CHUNK, CHUNK)],
                         sb[k], ss[k])
        pltpu.async_copy(dst_hbm.at[pl.ds(base0 + c * CHUNK, CHUNK)],
                         db[k], si[k])

    def wait_idx_src(sb, ss, k):
        pltpu.make_async_copy(src_hbm.at[pl.ds(0, CHUNK)],
                              sb[k], ss[k]).wait()

    def wait_idx_dst(db, si, k):
        pltpu.make_async_copy(dst_hbm.at[pl.ds(0, CHUNK)],
                              db[k], si[k]).wait()

    def issue_gather(sb, k):
        pltpu.async_copy(h_hbm.at[sb[k]], rbufs[k], gsems[k])

    def wait_gather(sb, k):
        pltpu.make_async_copy(h_hbm.at[sb[k]], rbufs[k], gsems[k]).wait()

    def scatter(db, k):
        pltpu.sync_copy(rbufs[k], agg_shared.at[db[k]], add=True)

    def half_round(r, cur, nxt):
        sb, db, ss, si = cur
        nsb, _, nss, _ = nxt
        c0 = r * NBUF
        for k in range(NBUF):
            wait_gather(sb, k)
            wait_idx_dst(db, si, k)
            scatter(db, k)
            issue_idx(c0 + 2 * NBUF + k, sb, db, ss, si, k)
            wait_idx_src(nsb, nss, k)
            issue_gather(nsb, k)

    A = (sbufsA, dbufsA, ssemsA, isemsA)
    B = (sbufsB, dbufsB, ssemsB, isemsB)

    for k in range(NBUF):
        issue_idx(k, sbufsA, dbufsA, ssemsA, isemsA, k)
        issue_idx(NBUF + k, sbufsB, dbufsB, ssemsB, isemsB, k)
    for k in range(NBUF):
        wait_idx_src(sbufsA, ssemsA, k)
        issue_gather(sbufsA, k)

    def body(t, carry):
        half_round(2 * t, A, B)
        half_round(2 * t + 1, B, A)
        return carry
    lax.fori_loop(0, (QUADS - 3) // 2, body, 0)

    r = QUADS - 3
    for k in range(NBUF):
        wait_gather(sbufsA, k)
        wait_idx_dst(dbufsA, isemsA, k)
        scatter(dbufsA, k)
        issue_idx(r * NBUF + 2 * NBUF + k, sbufsA, dbufsA, ssemsA, isemsA, k)
        wait_idx_src(sbufsB, ssemsB, k)
        issue_gather(sbufsB, k)
    r = QUADS - 2
    for k in range(NBUF):
        wait_gather(sbufsB, k)
        wait_idx_dst(dbufsB, isemsB, k)
        scatter(dbufsB, k)
        wait_idx_src(sbufsA, ssemsA, k)
        issue_gather(sbufsA, k)

    for k in range(NBUF):
        wait_gather(sbufsA, k)
        wait_idx_dst(dbufsA, isemsA, k)
        scatter(dbufsA, k)
        if k == 0 and REM_STEPS:
            issue_idx(STEPS - 1, sbufsB, dbufsB, ssemsB, isemsB, 0)
    if REM_STEPS:
        wait_idx_src(sbufsB, ssemsB, 0)
        issue_gather(sbufsB, 0)
        wait_gather(sbufsB, 0)
        wait_idx_dst(dbufsB, isemsB, 0)
        scatter(dbufsB, 0)

    plsc.subcore_barrier()
    _export_shared(cid, sid, agg_shared, agg_out)


def _deg_phase(dst_hbm, deg_out, cid, sid, dbufsA, dbufsB,
               isemsA, isemsB, ones_buf, deg_shared):
    base0 = cid * (NS * EPT) + sid * EPT

    def issue_idx(c, db, si, k):
        pltpu.async_copy(dst_hbm.at[pl.ds(base0 + c * CHUNK, CHUNK)],
                         db[k], si[k])

    def wait_idx(db, si, k):
        pltpu.make_async_copy(dst_hbm.at[pl.ds(0, CHUNK)],
                              db[k], si[k]).wait()

    def scatter(db, k):
        pltpu.sync_copy(ones_buf, deg_shared.at[db[k]], add=True)

    for k in range(NBUF):
        issue_idx(k, dbufsA, isemsA, k)
        issue_idx(NBUF + k, dbufsB, isemsB, k)

    def half_round(r, db, si, ss_db, ss_si):
        c0 = r * NBUF
        for k in range(NBUF):
            wait_idx(db, si, k)
            scatter(db, k)
            issue_idx(c0 + 2 * NBUF + k, db, si, k)

    def body(t, carry):
        half_round(2 * t, dbufsA, isemsA, None, None)
        half_round(2 * t + 1, dbufsB, isemsB, None, None)
        return carry
    lax.fori_loop(0, (QUADS - 3) // 2, body, 0)

    r = QUADS - 3
    for k in range(NBUF):
        wait_idx(dbufsA, isemsA, k)
        scatter(dbufsA, k)
        issue_idx(r * NBUF + 2 * NBUF + k, dbufsA, isemsA, k)
    for k in range(NBUF):
        wait_idx(dbufsB, isemsB, k)
        scatter(dbufsB, k)
    for k in range(NBUF):
        wait_idx(dbufsA, isemsA, k)
        scatter(dbufsA, k)
        if k == 0 and REM_STEPS:
            issue_idx(STEPS - 1, dbufsB, isemsB, 0)
    if REM_STEPS:
        wait_idx(dbufsB, isemsB, 0)
        scatter(dbufsB, 0)

    plsc.subcore_barrier()
    _export_shared(cid, sid, deg_shared, deg_out)


def _unpack_sc(sc):
    sbufsA = sc[0:4]
    sbufsB = sc[4:8]
    dbufsA = sc[8:12]
    dbufsB = sc[12:16]
    rbufs = sc[16:20]
    shared = sc[20]
    ssemsA = sc[21:25]
    ssemsB = sc[25:29]
    isemsA = sc[29:33]
    isemsB = sc[33:37]
    gsems = sc[37:41]
    return (sbufsA, sbufsB, dbufsA, dbufsB, rbufs, shared,
            ssemsA, ssemsB, isemsA, isemsB, gsems)


def _agg_body(h_hbm, src_hbm, dst_hbm, agg_out, *sc):
    (sbufsA, sbufsB, dbufsA, dbufsB, rbufs, shared,
     ssemsA, ssemsB, isemsA, isemsB, gsems) = _unpack_sc(sc)
    cid = lax.axis_index("c")
    sid = lax.axis_index("s")
    _agg_phase(h_hbm, src_hbm, dst_hbm, agg_out, cid, sid,
               sbufsA, sbufsB, dbufsA, dbufsB, rbufs,
               ssemsA, ssemsB, isemsA, isemsB, gsems, shared)


def _agg_deg_body(h_hbm, src_hbm, dst_hbm, agg_out, deg_out, *sc):
    (sbufsA, sbufsB, dbufsA, dbufsB, rbufs, shared,
     ssemsA, ssemsB, isemsA, isemsB, gsems) = _unpack_sc(sc)
    cid = lax.axis_index("c")
    sid = lax.axis_index("s")
    _agg_phase(h_hbm, src_hbm, dst_hbm, agg_out, cid, sid,
               sbufsA, sbufsB, dbufsA, dbufsB, rbufs,
               ssemsA, ssemsB, isemsA, isemsB, gsems, shared)
    plsc.subcore_barrier()
    _fill_buf(rbufs[0], jnp.zeros((LANES,), jnp.float32))
    _zero_shared(sid, rbufs[0], shared)
    _fill_buf(rbufs[0], jnp.ones((LANES,), jnp.float32))
    plsc.subcore_barrier()
    _deg_phase(dst_hbm, deg_out, cid, sid, dbufsA, dbufsB,
               isemsA, isemsB, rbufs[0], shared)


_SC_MESH = plsc.VectorSubcoreMesh(core_axis_name="c", subcore_axis_name="s")

_SC_SCRATCH = (pltpu.VMEM((CHUNK,), jnp.int32),) * 16 + (
    pltpu.VMEM((CHUNK, D), jnp.float32),
    pltpu.VMEM((CHUNK, D), jnp.float32),
    pltpu.VMEM((CHUNK, D), jnp.float32),
    pltpu.VMEM((CHUNK, D), jnp.float32),
    pltpu.VMEM_SHARED((N, D), jnp.float32),
) + (pltpu.SemaphoreType.DMA,) * 20

_agg_pass = pl.kernel(
    _agg_body,
    out_type=jax.ShapeDtypeStruct((NC, N, D), jnp.float32),
    mesh=_SC_MESH,
    scratch_types=_SC_SCRATCH,
)

_agg_deg_pass = pl.kernel(
    _agg_deg_body,
    out_type=(jax.ShapeDtypeStruct((NC, N, D), jnp.float32),
              jax.ShapeDtypeStruct((NC, N, D), jnp.float32)),
    mesh=_SC_MESH,
    scratch_types=_SC_SCRATCH,
)


def _layer_math(hprev, agg2, deg2, W_l, b_l, W_r, gamma, beta):
    agg = agg2[0] + agg2[1]
    deg = deg2[0] + deg2[1]
    mean = agg / jnp.maximum(deg, 1.0)
    pre = (jnp.dot(mean, W_l, preferred_element_type=jnp.float32)
           + jnp.dot(hprev, W_r, preferred_element_type=jnp.float32)
           + b_l)
    mu = jnp.mean(pre, axis=0, keepdims=True)
    cen = pre - mu
    var = jnp.mean(cen * cen, axis=0, keepdims=True)
    return jnp.maximum(cen * lax.rsqrt(var + EPS) * gamma + beta, 0.0)


def _layer0_body(h_ref, agg_ref, deg_ref, wl_ref, bl_ref, wr_ref, g_ref, be_ref,
                 out_ref):
    out_ref[...] = _layer_math(h_ref[...], agg_ref[...], deg_ref[...],
                               wl_ref[...], bl_ref[...], wr_ref[...],
                               g_ref[...], be_ref[...])


def _layer1_body(h_ref, agg_ref, deg_ref, wl_ref, bl_ref, wr_ref, g_ref, be_ref,
                 batch_ref, out_ref):
    h2 = _layer_math(h_ref[...], agg_ref[...], deg_ref[...],
                     wl_ref[...], bl_ref[...], wr_ref[...],
                     g_ref[...], be_ref[...])
    gids = lax.broadcasted_iota(jnp.int32, (G, N), 0)
    onehot = jnp.where(gids == batch_ref[...], 1.0, 0.0)
    out_ref[...] = lax.dot_general(onehot, h2, (((1,), (0,)), ((), ())),
                                   preferred_element_type=jnp.float32)


_layer0 = pl.pallas_call(
    _layer0_body,
    out_shape=jax.ShapeDtypeStruct((N, D), jnp.float32),
)

_layer1 = pl.pallas_call(
    _layer1_body,
    out_shape=jax.ShapeDtypeStruct((G, D), jnp.float32),
)


def kernel(x, edge_index, batch, W_l0, b_l0, W_r0, gamma0, beta0,
           W_l1, b_l1, W_r1, gamma1, beta1):
    src = edge_index[0]
    dst = edge_index[1]
    batch2d = batch.reshape(1, N)

    agg0, deg2 = _agg_deg_pass(x, src, dst)
    h1 = _layer0(x, agg0, deg2, W_l0, b_l0.reshape(1, D), W_r0,
                 gamma0.reshape(1, D), beta0.reshape(1, D))
    agg1 = _agg_pass(h1, src, dst)
    out = _layer1(h1, agg1, deg2, W_l1, b_l1.reshape(1, D), W_r1,
                  gamma1.reshape(1, D), beta1.reshape(1, D), batch2d)
    return out


# speedup vs baseline: 12.5950x; 1.0008x over previous
import jax
import jax.numpy as jnp
from jax import lax
from jax.experimental import pallas as pl
from jax.experimental.pallas import tpu as pltpu
from jax.experimental.pallas import tpu_sc as plsc

N = 10000
E = 320000
D = 128
G = 64
EPS = 1e-5

NC = 2
NS = 16
LANES = 16
CHUNK = 80
EPT = E // (NC * NS)
STEPS = EPT // CHUNK
RPT = 624
TAIL = N - NS * RPT
REM = RPT % CHUNK


def _fill_buf(buf, vec):
    def body(i, carry):
        for c in range(D // LANES):
            buf[i, pl.ds(c * LANES, LANES)] = vec
        return carry
    lax.fori_loop(0, CHUNK, body, 0)


def _zero_shared(sid, buf, shared):
    for j in range(RPT // CHUNK):
        pltpu.sync_copy(buf, shared.at[pl.ds(sid * RPT + j * CHUNK, CHUNK)])
    if REM:
        pltpu.sync_copy(buf.at[pl.ds(0, REM)],
                        shared.at[pl.ds(sid * RPT + (RPT // CHUNK) * CHUNK, REM)])

    @pl.when(sid == NS - 1)
    def _zero_tail():
        pltpu.sync_copy(buf.at[pl.ds(0, TAIL)], shared.at[pl.ds(NS * RPT, TAIL)])


def _export_shared(cid, sid, shared, out_hbm):
    pltpu.sync_copy(shared.at[pl.ds(sid * RPT, RPT)],
                    out_hbm.at[cid, pl.ds(sid * RPT, RPT)])

    @pl.when(sid == NS - 1)
    def _export_tail():
        pltpu.sync_copy(shared.at[pl.ds(NS * RPT, TAIL)],
                        out_hbm.at[cid, pl.ds(NS * RPT, TAIL)])


NBUF = 4
QUADS = STEPS // NBUF
REM_STEPS = STEPS % NBUF


def _agg_phase(h_hbm, src_hbm, dst_hbm, agg_out, cid, sid,
               sbufsA, sbufsB, dbufsA, dbufsB, rbufs,
               ssemsA, ssemsB, isemsA, isemsB, gsems, agg_shared):
    base0 = cid * (NS * EPT) + sid * EPT

    _fill_buf(rbufs[0], jnp.zeros((LANES,), jnp.float32))
    _zero_shared(sid, rbufs[0], agg_shared)
    plsc.subcore_barrier()

    def issue_idx(c, sb, db, ss, si, k):
        pltpu.async_copy(src_hbm.at[pl.ds(base0 + c * CHUNK, CHUNK)],
                         sb[k], ss[k])
        pltpu.async_copy(dst_hbm.at[pl.ds(base0 + c * CHUNK, CHUNK)],
                         db[k], si[k])

    def wait_idx_src(sb, ss, k):
        pltpu.make_async_copy(src_hbm.at[pl.ds(0, CHUNK)],
                              sb[k], ss[k]).wait()

    def wait_idx_dst(db, si, k):
        pltpu.make_async_copy(dst_hbm.at[pl.ds(0, CHUNK)],
                              db[k], si[k]).wait()

    def issue_gather(sb, k):
        pltpu.async_copy(h_hbm.at[sb[k]], rbufs[k], gsems[k])

    def wait_gather(sb, k):
        pltpu.make_async_copy(h_hbm.at[sb[k]], rbufs[k], gsems[k]).wait()

    def scatter(db, k):
        pltpu.sync_copy(rbufs[k], agg_shared.at[db[k]], add=True)

    def half_round(r, cur, nxt):
        sb, db, ss, si = cur
        nsb, _, nss, _ = nxt
        c0 = r * NBUF
        for k in range(NBUF):
            wait_gather(sb, k)
            wait_idx_dst(db, si, k)
            scatter(db, k)
            issue_idx(c0 + 2 * NBUF + k, sb, db, ss, si, k)
            wait_idx_src(nsb, nss, k)
            issue_gather(nsb, k)

    A = (sbufsA, dbufsA, ssemsA, isemsA)
    B = (sbufsB, dbufsB, ssemsB, isemsB)

    for k in range(NBUF):
        issue_idx(k, sbufsA, dbufsA, ssemsA, isemsA, k)
        issue_idx(NBUF + k, sbufsB, dbufsB, ssemsB, isemsB, k)
    for k in range(NBUF):
        wait_idx_src(sbufsA, ssemsA, k)
        issue_gather(sbufsA, k)

    def body(t, carry):
        half_round(2 * t, A, B)
        half_round(2 * t + 1, B, A)
        return carry
    lax.fori_loop(0, (QUADS - 3) // 2, body, 0)

    r = QUADS - 3
    for k in range(NBUF):
        wait_gather(sbufsA, k)
        wait_idx_dst(dbufsA, isemsA, k)
        scatter(dbufsA, k)
        issue_idx(r * NBUF + 2 * NBUF + k, sbufsA, dbufsA, ssemsA, isemsA, k)
        wait_idx_src(sbufsB, ssemsB, k)
        issue_gather(sbufsB, k)
    r = QUADS - 2
    for k in range(NBUF):
        wait_gather(sbufsB, k)
        wait_idx_dst(dbufsB, isemsB, k)
        scatter(dbufsB, k)
        wait_idx_src(sbufsA, ssemsA, k)
        issue_gather(sbufsA, k)

    for k in range(NBUF):
        wait_gather(sbufsA, k)
        wait_idx_dst(dbufsA, isemsA, k)
        scatter(dbufsA, k)
        if k == 0 and REM_STEPS:
            issue_idx(STEPS - 1, sbufsB, dbufsB, ssemsB, isemsB, 0)
    if REM_STEPS:
        wait_idx_src(sbufsB, ssemsB, 0)
        issue_gather(sbufsB, 0)
        wait_gather(sbufsB, 0)
        wait_idx_dst(dbufsB, isemsB, 0)
        scatter(dbufsB, 0)

    plsc.subcore_barrier()
    _export_shared(cid, sid, agg_shared, agg_out)


def _deg_phase(dst_hbm, deg_out, cid, sid, dbufsA, dbufsB,
               isemsA, isemsB, ones_buf, deg_shared):
    base0 = cid * (NS * EPT) + sid * EPT

    def issue_idx(c, db, si, k):
        pltpu.async_copy(dst_hbm.at[pl.ds(base0 + c * CHUNK, CHUNK)],
                         db[k], si[k])

    def wait_idx(db, si, k):
        pltpu.make_async_copy(dst_hbm.at[pl.ds(0, CHUNK)],
                              db[k], si[k]).wait()

    def scatter(db, k):
        pltpu.sync_copy(ones_buf, deg_shared.at[db[k]], add=True)

    for k in range(NBUF):
        issue_idx(k, dbufsA, isemsA, k)
        issue_idx(NBUF + k, dbufsB, isemsB, k)

    def half_round(r, db, si, ss_db, ss_si):
        c0 = r * NBUF
        for k in range(NBUF):
            wait_idx(db, si, k)
            scatter(db, k)
            issue_idx(c0 + 2 * NBUF + k, db, si, k)

    def body(t, carry):
        half_round(2 * t, dbufsA, isemsA, None, None)
        half_round(2 * t + 1, dbufsB, isemsB, None, None)
        return carry
    lax.fori_loop(0, (QUADS - 3) // 2, body, 0)

    r = QUADS - 3
    for k in range(NBUF):
        wait_idx(dbufsA, isemsA, k)
        scatter(dbufsA, k)
        issue_idx(r * NBUF + 2 * NBUF + k, dbufsA, isemsA, k)
    for k in range(NBUF):
        wait_idx(dbufsB, isemsB, k)
        scatter(dbufsB, k)
    for k in range(NBUF):
        wait_idx(dbufsA, isemsA, k)
        scatter(dbufsA, k)
        if k == 0 and REM_STEPS:
            issue_idx(STEPS - 1, dbufsB, isemsB, 0)
    if REM_STEPS:
        wait_idx(dbufsB, isemsB, 0)
        scatter(dbufsB, 0)

    plsc.subcore_barrier()
    _export_shared(cid, sid, deg_shared, deg_out)


def _unpack_sc(sc):
    sbufsA = sc[0:4]
    sbufsB = sc[4:8]
    dbufsA = sc[8:12]
    dbufsB = sc[12:16]
    rbufs = sc[16:20]
    shared = sc[20]
    ssemsA = sc[21:25]
    ssemsB = sc[25:29]
    isemsA = sc[29:33]
    isemsB = sc[33:37]
    gsems = sc[37:41]
    return (sbufsA, sbufsB, dbufsA, dbufsB, rbufs, shared,
            ssemsA, ssemsB, isemsA, isemsB, gsems)


def _agg_body(h_hbm, src_hbm, dst_hbm, agg_out, *sc):
    (sbufsA, sbufsB, dbufsA, dbufsB, rbufs, shared,
     ssemsA, ssemsB, isemsA, isemsB, gsems) = _unpack_sc(sc)
    cid = lax.axis_index("c")
    sid = lax.axis_index("s")
    _agg_phase(h_hbm, src_hbm, dst_hbm, agg_out, cid, sid,
               sbufsA, sbufsB, dbufsA, dbufsB, rbufs,
               ssemsA, ssemsB, isemsA, isemsB, gsems, shared)


def _agg_deg_body(h_hbm, src_hbm, dst_hbm, agg_out, deg_out, *sc):
    (sbufsA, sbufsB, dbufsA, dbufsB, rbufs, shared,
     ssemsA, ssemsB, isemsA, isemsB, gsems) = _unpack_sc(sc)
    cid = lax.axis_index("c")
    sid = lax.axis_index("s")
    _agg_phase(h_hbm, src_hbm, dst_hbm, agg_out, cid, sid,
               sbufsA, sbufsB, dbufsA, dbufsB, rbufs,
               ssemsA, ssemsB, isemsA, isemsB, gsems, shared)
    plsc.subcore_barrier()
    _fill_buf(rbufs[0], jnp.zeros((LANES,), jnp.float32))
    _zero_shared(sid, rbufs[0], shared)
    _fill_buf(rbufs[0], jnp.ones((LANES,), jnp.float32))
    plsc.subcore_barrier()
    _deg_phase(dst_hbm, deg_out, cid, sid, dbufsA, dbufsB,
               isemsA, isemsB, rbufs[0], shared)


_SC_MESH = plsc.VectorSubcoreMesh(core_axis_name="c", subcore_axis_name="s")

_SC_SCRATCH = (pltpu.VMEM((CHUNK,), jnp.int32),) * 16 + (
    pltpu.VMEM((CHUNK, D), jnp.float32),
    pltpu.VMEM((CHUNK, D), jnp.float32),
    pltpu.VMEM((CHUNK, D), jnp.float32),
    pltpu.VMEM((CHUNK, D), jnp.float32),
    pltpu.VMEM_SHARED((N, D), jnp.float32),
) + (pltpu.SemaphoreType.DMA,) * 20

_agg_pass = pl.kernel(
    _agg_body,
    out_type=jax.ShapeDtypeStruct((NC, N, D), jnp.float32),
    mesh=_SC_MESH,
    scratch_types=_SC_SCRATCH,
)

_agg_deg_pass = pl.kernel(
    _agg_deg_body,
    out_type=(jax.ShapeDtypeStruct((NC, N, D), jnp.float32),
              jax.ShapeDtypeStruct((NC, N, D), jnp.float32)),
    mesh=_SC_MESH,
    scratch_types=_SC_SCRATCH,
)


def _layer_math(hprev, agg2, deg2, W_l, b_l, W_r, gamma, beta):
    agg = agg2[0] + agg2[1]
    deg = deg2[0] + deg2[1]
    mean = agg / jnp.maximum(deg, 1.0)
    pre = (jnp.dot(mean, W_l, preferred_element_type=jnp.float32)
           + jnp.dot(hprev, W_r, preferred_element_type=jnp.float32)
           + b_l)
    ones_row = jnp.ones((1, N), jnp.float32)
    mu = lax.dot_general(ones_row, pre, (((1,), (0,)), ((), ())),
                         preferred_element_type=jnp.float32) * (1.0 / N)
    cen = pre - mu
    var = lax.dot_general(ones_row, cen * cen, (((1,), (0,)), ((), ())),
                          preferred_element_type=jnp.float32) * (1.0 / N)
    return jnp.maximum(cen * lax.rsqrt(var + EPS) * gamma + beta, 0.0)


def _layer0_body(h_ref, agg_ref, deg_ref, wl_ref, bl_ref, wr_ref, g_ref, be_ref,
                 out_ref):
    out_ref[...] = _layer_math(h_ref[...], agg_ref[...], deg_ref[...],
                               wl_ref[...], bl_ref[...], wr_ref[...],
                               g_ref[...], be_ref[...])


def _layer1_body(h_ref, agg_ref, deg_ref, wl_ref, bl_ref, wr_ref, g_ref, be_ref,
                 batch_ref, out_ref):
    h2 = _layer_math(h_ref[...], agg_ref[...], deg_ref[...],
                     wl_ref[...], bl_ref[...], wr_ref[...],
                     g_ref[...], be_ref[...])
    gids = lax.broadcasted_iota(jnp.int32, (G, N), 0)
    onehot = jnp.where(gids == batch_ref[...], 1.0, 0.0)
    out_ref[...] = lax.dot_general(onehot, h2, (((1,), (0,)), ((), ())),
                                   preferred_element_type=jnp.float32)


_layer0 = pl.pallas_call(
    _layer0_body,
    out_shape=jax.ShapeDtypeStruct((N, D), jnp.float32),
)

_layer1 = pl.pallas_call(
    _layer1_body,
    out_shape=jax.ShapeDtypeStruct((G, D), jnp.float32),
)


def kernel(x, edge_index, batch, W_l0, b_l0, W_r0, gamma0, beta0,
           W_l1, b_l1, W_r1, gamma1, beta1):
    src = edge_index[0]
    dst = edge_index[1]
    batch2d = batch.reshape(1, N)

    agg0, deg2 = _agg_deg_pass(x, src, dst)
    h1 = _layer0(x, agg0, deg2, W_l0, b_l0.reshape(1, D), W_r0,
                 gamma0.reshape(1, D), beta0.reshape(1, D))
    agg1 = _agg_pass(h1, src, dst)
    out = _layer1(h1, agg1, deg2, W_l1, b_l1.reshape(1, D), W_r1,
                  gamma1.reshape(1, D), beta1.reshape(1, D), batch2d)
    return out
